# baseline (speedup 1.0000x reference)
import jax
import jax.numpy as jnp
import numpy as np
from jax.experimental import pallas as pl
from jax.experimental.pallas import tpu as pltpu

B = 1024
NUM_NUMERIC = 13
NUM_CAT = 26
CARD = 1000
M = NUM_NUMERIC + NUM_CAT
D = 10
F = NUM_NUMERIC + NUM_CAT * CARD
H = 200
HP = 256
EP = 128
ED = M * D
GRP = (12, 12, 2)

BT1 = 128
BT2 = 256


def _embed_body(x_ref, wlr_ref, wnumd_ref, g0_ref, g1_ref, g2_ref, e_ref):
    f32 = jnp.float32
    dg = lambda a, b: jax.lax.dot_general(
        a, b, (((1,), (1,)), ((), ())), preferred_element_type=f32)

    xn = x_ref[:, 0:NUM_NUMERIC]
    e_ref[:, 0:NUM_NUMERIC * D] = jnp.dot(
        xn, wnumd_ref[...], preferred_element_type=f32)
    lin = dg(xn, wlr_ref[:, 0:NUM_NUMERIC])

    col = NUM_NUMERIC * D
    lo = NUM_NUMERIC
    for g_ref, nf in zip((g0_ref, g1_ref, g2_ref), GRP):
        k = nf * CARD
        xg = x_ref[:, lo:lo + k]
        e_ref[:, col:col + nf * D] = jnp.dot(
            xg, g_ref[...], preferred_element_type=f32)
        lin = lin + dg(xg, wlr_ref[:, lo:lo + k])
        lo += k
        col += nf * D
    e_ref[:, ED:ED + 1] = lin


def _cin_dnn_body(e_ref, w0_ref, b0_ref, w1_ref, b1_ref, w2_ref, b2_ref,
                  clw_ref, dw0_ref, db0_ref, dw1_ref, db1_ref, dlw_ref,
                  perm_ref, cb_ref, out_ref, edm_scr):
    f32 = jnp.float32
    e390 = e_ref[:, 0:ED]
    lin = e_ref[:, ED:ED + 1]

    edm_scr[...] = jnp.dot(e390, perm_ref[...], preferred_element_type=f32)

    def step(d, carry):
        p0, p1, p2 = carry
        ed = edm_scr[:, pl.ds(d * EP, EP)]
        ei = ed[:, :M, None]
        z0 = (ei * ed[:, None, :]).reshape(BT2, M * EP)
        c1 = jnp.dot(z0, w0_ref[...],
                     preferred_element_type=f32) + b0_ref[...]
        z1 = (ei * c1[:, None, :]).reshape(BT2, M * HP)
        c2 = jnp.dot(z1, w1_ref[...],
                     preferred_element_type=f32) + b1_ref[...]
        z2 = (ei * c2[:, None, :]).reshape(BT2, M * HP)
        c3 = jnp.dot(z2, w2_ref[...],
                     preferred_element_type=f32) + b2_ref[...]
        return (p0 + c1, p1 + c2, p2 + c3)

    zp = jnp.zeros((BT2, HP), f32)
    p0, p1, p2 = jax.lax.fori_loop(0, D, step, (zp, zp, zp))
    pooled = jnp.concatenate([p0, p1, p2], axis=1)
    cin = jnp.dot(pooled, clw_ref[...], preferred_element_type=f32)

    h = jnp.maximum(jnp.dot(e390, dw0_ref[...], preferred_element_type=f32)
                    + db0_ref[...], 0.0)
    h = jnp.maximum(jnp.dot(h, dw1_ref[...], preferred_element_type=f32)
                    + db1_ref[...], 0.0)
    dnn = jnp.dot(h, dlw_ref[...], preferred_element_type=f32)

    out_ref[...] = jax.nn.sigmoid(lin + cin + dnn + cb_ref[0, 0])


@jax.jit
def kernel(x, w_lin, b_lin, W_num, W_cat,
           cin_w0, cin_b0, cin_w1, cin_b1, cin_w2, cin_b2,
           cin_lin_w, cin_lin_b,
           dnn_w0, dnn_b0, dnn_w1, dnn_b1, dnn_lin_w, dnn_lin_b, pred_b):
    f32 = jnp.float32

    rows = jnp.arange(NUM_NUMERIC)
    wnumd = jnp.zeros((NUM_NUMERIC, NUM_NUMERIC * D), f32).at[
        rows[:, None], rows[:, None] * D + jnp.arange(D)[None, :]].set(W_num)

    def blockdiag(ws):
        nf = ws.shape[0]
        out = jnp.zeros((nf * CARD, nf * D), f32)
        r = jnp.arange(nf * CARD)
        c = (r // CARD)[:, None] * D + jnp.arange(D)[None, :]
        return out.at[r[:, None], c].set(ws.reshape(nf * CARD, D))

    g0 = blockdiag(W_cat[0:12])
    g1 = blockdiag(W_cat[12:24])
    g2 = blockdiag(W_cat[24:26])

    wlin_row = w_lin.reshape(1, F)

    def repad(w, h_prev, p):
        w3 = w.reshape(M, h_prev, -1)
        w3 = jnp.pad(w3, ((0, 0), (0, p - h_prev), (0, HP - w.shape[1])))
        return w3.reshape(M * p, HP)

    w0p = repad(cin_w0, M, EP)
    w1p = repad(cin_w1, H, HP)
    w2p = repad(cin_w2, H, HP)
    b0p = jnp.pad(cin_b0, (0, HP - H)).reshape(1, HP)
    b1p = jnp.pad(cin_b1, (0, HP - H)).reshape(1, HP)
    b2p = jnp.pad(cin_b2, (0, HP - H)).reshape(1, HP)
    clwp = jnp.pad(cin_lin_w.reshape(3, H, 1),
                   ((0, 0), (0, HP - H), (0, 0))).reshape(3 * HP, 1)

    pf = np.zeros((ED, D * EP), np.float32)
    fidx = np.repeat(np.arange(M), D)
    didx = np.tile(np.arange(D), M)
    pf[np.arange(ED), didx * EP + fidx] = 1.0
    perm = jnp.asarray(pf)

    cb = (b_lin + cin_lin_b + dnn_lin_b + pred_b).reshape(1, 1)
    db0 = dnn_b0.reshape(1, -1)
    db1 = dnn_b1.reshape(1, -1)

    wspec = pl.BlockSpec(memory_space=pltpu.VMEM)

    e_all = pl.pallas_call(
        _embed_body,
        out_shape=jax.ShapeDtypeStruct((B, ED + 1), f32),
        grid=(B // BT1,),
        in_specs=[pl.BlockSpec((BT1, F), lambda i: (i, 0)),
                  wspec, wspec, wspec, wspec, wspec],
        out_specs=pl.BlockSpec((BT1, ED + 1), lambda i: (i, 0)),
        compiler_params=pltpu.CompilerParams(
            dimension_semantics=("parallel",),
            vmem_limit_bytes=60 * 1024 * 1024,
        ),
        name="xdeepfm_embed",
    )(x, wlin_row, wnumd, g0, g1, g2)

    out = pl.pallas_call(
        _cin_dnn_body,
        out_shape=jax.ShapeDtypeStruct((B, 1), f32),
        grid=(B // BT2,),
        in_specs=[pl.BlockSpec((BT2, ED + 1), lambda i: (i, 0)),
                  wspec, wspec, wspec, wspec, wspec, wspec,
                  wspec,
                  wspec, wspec, wspec, wspec, wspec,
                  wspec,
                  pl.BlockSpec(memory_space=pltpu.SMEM)],
        out_specs=pl.BlockSpec((BT2, 1), lambda i: (i, 0)),
        scratch_shapes=[pltpu.VMEM((BT2, D * EP), f32)],
        compiler_params=pltpu.CompilerParams(
            dimension_semantics=("parallel",),
            vmem_limit_bytes=60 * 1024 * 1024,
        ),
        name="xdeepfm_cin_dnn",
    )(e_all, w0p, b0p, w1p, b1p, w2p, b2p, clwp,
      dnn_w0, db0, dnn_w1, db1, dnn_lin_w, perm, cb)
    return out


# speedup vs baseline: 2.8119x; 2.8119x over previous
import jax
import jax.numpy as jnp
import numpy as np
from jax.experimental import pallas as pl
from jax.experimental.pallas import tpu as pltpu

B = 1024
NUM_NUMERIC = 13
NUM_CAT = 26
CARD = 1000
M = NUM_NUMERIC + NUM_CAT
D = 10
F = NUM_NUMERIC + NUM_CAT * CARD
H = 200
HP = 256
EP = 128
ED = M * D
GRP = (12, 12, 2)

BT1 = 128
BT2 = 256

def _split_hi(v):
    bits = jax.lax.bitcast_convert_type(v, jnp.uint32)
    hi = jax.lax.bitcast_convert_type(
        bits & np.uint32(0xFFFF0000), jnp.float32)
    return hi, v - hi


def _embed_body(x_ref, wnumd_ref, g0_ref, g1_ref, g2_ref, e_ref):
    f32 = jnp.float32

    xn = x_ref[:, 0:NUM_NUMERIC]
    en = jnp.dot(xn, wnumd_ref[...], preferred_element_type=f32,
                 precision=jax.lax.Precision.HIGHEST)
    e_ref[:, 0:NUM_NUMERIC * D] = en[:, 0:NUM_NUMERIC * D]
    lin = en[:, NUM_NUMERIC * D:NUM_NUMERIC * D + 1]

    col = NUM_NUMERIC * D
    lo = NUM_NUMERIC
    for g_ref, nf in zip((g0_ref, g1_ref, g2_ref), GRP):
        k = nf * CARD
        eg = jnp.dot(x_ref[:, lo:lo + k], g_ref[...],
                     preferred_element_type=f32)
        e_ref[:, col:col + nf * D] = eg[:, 0:nf * D]
        lin = lin + eg[:, nf * D:nf * D + 1]
        lo += k
        col += nf * D
    e_ref[:, ED:ED + 1] = lin


def _cin_dnn_body(e_ref, w0_ref, b0_ref, w1_ref, b1_ref, w2_ref, b2_ref,
                  clw_ref, dw0_ref, db0_ref, dw1_ref, db1_ref, dlw_ref,
                  perm_ref, cb_ref, out_ref, edm_scr):
    f32 = jnp.float32
    e390 = e_ref[:, 0:ED]
    lin = e_ref[:, ED:ED + 1]

    ehi, elo = _split_hi(e390)
    edm_scr[...] = (
        jnp.dot(ehi, perm_ref[...], preferred_element_type=f32)
        + jnp.dot(elo, perm_ref[...], preferred_element_type=f32))

    def step(d, carry):
        p0, p1, p2 = carry
        ed = edm_scr[:, pl.ds(d * EP, EP)]
        ei = ed[:, :M, None]
        z0 = (ei * ed[:, None, :]).reshape(BT2, M * EP)
        c1 = jnp.dot(z0, w0_ref[...],
                     preferred_element_type=f32) + b0_ref[...]
        z1 = (ei * c1[:, None, :]).reshape(BT2, M * HP)
        c2 = jnp.dot(z1, w1_ref[...],
                     preferred_element_type=f32) + b1_ref[...]
        z2 = (ei * c2[:, None, :]).reshape(BT2, M * HP)
        c3 = jnp.dot(z2, w2_ref[...],
                     preferred_element_type=f32) + b2_ref[...]
        return (p0 + c1, p1 + c2, p2 + c3)

    zp = jnp.zeros((BT2, HP), f32)
    p0, p1, p2 = jax.lax.fori_loop(0, D, step, (zp, zp, zp))
    pooled = jnp.concatenate([p0, p1, p2], axis=1)
    cin = jnp.dot(pooled, clw_ref[...], preferred_element_type=f32)

    h = jnp.maximum(jnp.dot(e390, dw0_ref[...], preferred_element_type=f32)
                    + db0_ref[...], 0.0)
    h = jnp.maximum(jnp.dot(h, dw1_ref[...], preferred_element_type=f32)
                    + db1_ref[...], 0.0)
    dnn = jnp.dot(h, dlw_ref[...], preferred_element_type=f32)

    out_ref[...] = jax.nn.sigmoid(lin + cin + dnn + cb_ref[0, 0])


@jax.jit
def kernel(x, w_lin, b_lin, W_num, W_cat,
           cin_w0, cin_b0, cin_w1, cin_b1, cin_w2, cin_b2,
           cin_lin_w, cin_lin_b,
           dnn_w0, dnn_b0, dnn_w1, dnn_b1, dnn_lin_w, dnn_lin_b, pred_b):
    f32 = jnp.float32

    eye13 = jnp.eye(NUM_NUMERIC, dtype=f32)
    wnumd = (W_num[:, None, :] * eye13[:, :, None]
             ).transpose(1, 0, 2).reshape(NUM_NUMERIC, NUM_NUMERIC * D)
    wnumd = jnp.concatenate([wnumd, w_lin[0:NUM_NUMERIC]], axis=1)

    def blockdiag(ws, wl):
        nf = ws.shape[0]
        eye = jnp.eye(nf, dtype=f32)
        g = (ws[:, :, None, :] * eye[:, None, :, None]
             ).reshape(nf * CARD, nf * D)
        return jnp.concatenate([g, wl.reshape(nf * CARD, 1)], axis=1)

    g0 = blockdiag(W_cat[0:12], w_lin[13:12013])
    g1 = blockdiag(W_cat[12:24], w_lin[12013:24013])
    g2 = blockdiag(W_cat[24:26], w_lin[24013:26013])

    def repad(w, h_prev, p):
        w3 = w.reshape(M, h_prev, -1)
        w3 = jnp.pad(w3, ((0, 0), (0, p - h_prev), (0, HP - w.shape[1])))
        return w3.reshape(M * p, HP)

    w0p = repad(cin_w0, M, EP)
    w1p = repad(cin_w1, H, HP)
    w2p = repad(cin_w2, H, HP)
    b0p = jnp.pad(cin_b0, (0, HP - H)).reshape(1, HP)
    b1p = jnp.pad(cin_b1, (0, HP - H)).reshape(1, HP)
    b2p = jnp.pad(cin_b2, (0, HP - H)).reshape(1, HP)
    clwp = jnp.pad(cin_lin_w.reshape(3, H, 1),
                   ((0, 0), (0, HP - H), (0, 0))).reshape(3 * HP, 1)

    pf = np.zeros((ED, D * EP), np.float32)
    fidx = np.repeat(np.arange(M), D)
    didx = np.tile(np.arange(D), M)
    pf[np.arange(ED), didx * EP + fidx] = 1.0
    perm = jnp.asarray(pf)

    cb = (b_lin + cin_lin_b + dnn_lin_b + pred_b).reshape(1, 1)
    db0 = dnn_b0.reshape(1, -1)
    db1 = dnn_b1.reshape(1, -1)

    wspec = pl.BlockSpec(memory_space=pltpu.VMEM)

    e_all = pl.pallas_call(
        _embed_body,
        out_shape=jax.ShapeDtypeStruct((B, ED + 1), f32),
        grid=(B // BT1,),
        in_specs=[pl.BlockSpec((BT1, F), lambda i: (i, 0)),
                  wspec, wspec, wspec, wspec],
        out_specs=pl.BlockSpec((BT1, ED + 1), lambda i: (i, 0)),
        compiler_params=pltpu.CompilerParams(
            dimension_semantics=("parallel",),
            vmem_limit_bytes=60 * 1024 * 1024,
        ),
        name="xdeepfm_embed",
    )(x, wnumd, g0, g1, g2)

    out = pl.pallas_call(
        _cin_dnn_body,
        out_shape=jax.ShapeDtypeStruct((B, 1), f32),
        grid=(B // BT2,),
        in_specs=[pl.BlockSpec((BT2, ED + 1), lambda i: (i, 0)),
                  wspec, wspec, wspec, wspec, wspec, wspec,
                  wspec,
                  wspec, wspec, wspec, wspec, wspec,
                  wspec,
                  pl.BlockSpec(memory_space=pltpu.SMEM)],
        out_specs=pl.BlockSpec((BT2, 1), lambda i: (i, 0)),
        scratch_shapes=[pltpu.VMEM((BT2, D * EP), f32)],
        compiler_params=pltpu.CompilerParams(
            dimension_semantics=("parallel",),
            vmem_limit_bytes=60 * 1024 * 1024,
        ),
        name="xdeepfm_cin_dnn",
    )(e_all, w0p, b0p, w1p, b1p, w2p, b2p, clwp,
      dnn_w0, db0, dnn_w1, db1, dnn_lin_w, perm, cb)
    return out
